# Initial kernel scaffold; baseline (speedup 1.0000x reference)
import jax
import jax.numpy as jnp
from jax.experimental import pallas as pl


def kernel(x, edge_index, curr_vertex_index, Wc1, bc1, Wc2, bc2, Wc3, bc3, Wa1, ba1, Wa2, ba2, Wa3, ba3, Wk1, bk1, Wk2, bk2, Wk3, bk3, Wd1, bd1, Wd2, bd2, Wd3, bd3, Wd4, bd4):
    raise NotImplementedError("write your pallas kernel here")


# speedup vs baseline: 13.7552x; 13.7552x over previous
import functools

import jax
import jax.numpy as jnp
from jax import lax
from jax.experimental import pallas as pl
from jax.experimental.pallas import tpu as pltpu
from jax.experimental.pallas import tpu_sc as plsc

N = 1024
E = 32768
ALPHA = 0.1


def _make_count_kernel(num_cores, num_subcores):
    NW = num_cores * num_subcores
    EPW = E // NW
    CHUNK = 128
    NCHUNK = EPW // CHUNK
    WPT = (N * N) // num_subcores

    mesh = plsc.VectorSubcoreMesh(core_axis_name="c", subcore_axis_name="s")

    @functools.partial(
        pl.kernel,
        mesh=mesh,
        out_type=jax.ShapeDtypeStruct((num_cores, num_subcores, WPT), jnp.float32),
        scratch_types=[
            pltpu.VMEM((EPW,), jnp.int32),
            pltpu.VMEM((EPW,), jnp.int32),
            pltpu.VMEM((NCHUNK, CHUNK), jnp.int32),
            pltpu.VMEM((CHUNK,), jnp.float32),
            pltpu.VMEM_SHARED((N * N,), jnp.float32),
        ],
    )
    def count_kernel(src_hbm, dst_hbm, zero_hbm, out_hbm,
                     src_v, dst_v, idx_v, ones_v, c_sh):
        c = lax.axis_index("c")
        s = lax.axis_index("s")
        base = (c * num_subcores + s) * EPW

        pltpu.sync_copy(zero_hbm, c_sh.at[pl.ds(s * WPT, WPT)])
        pltpu.sync_copy(src_hbm.at[pl.ds(base, EPW)], src_v)
        pltpu.sync_copy(dst_hbm.at[pl.ds(base, EPW)], dst_v)

        for i in range(CHUNK // 16):
            ones_v[pl.ds(i * 16, 16)] = jnp.ones((16,), jnp.float32)
        for k in range(EPW // 16):
            sv = src_v[pl.ds(k * 16, 16)]
            dv = dst_v[pl.ds(k * 16, 16)]
            idx_v[(k * 16) // CHUNK, pl.ds((k * 16) % CHUNK, 16)] = dv * N + sv

        plsc.subcore_barrier()
        for j in range(NCHUNK):
            pltpu.sync_copy(ones_v, c_sh.at[idx_v.at[j]], add=True)
        plsc.subcore_barrier()
        pltpu.sync_copy(c_sh.at[pl.ds(s * WPT, WPT)], out_hbm.at[c, s])

    return count_kernel


def _lrelu(v):
    return jnp.where(v > 0, v, ALPHA * v)


def _gnn_body(cp, x_ref, oh_ref,
              wc1, bc1, wc2, bc2, wc3, bc3,
              wk1, bk1, wk2, bk2, wk3, bk3,
              wa1, ba1, wa2, ba2, wa3, ba3,
              probs_ref, g_ref, c_scr):
    c_scr[...] = cp[0] + cp[1]
    C = c_scr[...]
    deg = jnp.sum(C, axis=1, keepdims=True) + 1.0
    dinv = lax.rsqrt(deg)
    nbr = jnp.sum(C * oh_ref[...], axis=1, keepdims=True)

    def gcn(h, W, b):
        u = jnp.dot(h, W, preferred_element_type=jnp.float32)
        us = u * dinv
        agg = jnp.dot(C, us, preferred_element_type=jnp.float32) + us
        return agg * dinv + b

    h = jnp.tanh(gcn(x_ref[...], wc1[...], bc1[...]))
    h = jnp.tanh(gcn(h, wc2[...], bc2[...]))
    h = jnp.tanh(gcn(h, wc3[...], bc3[...]))
    s1 = _lrelu(jnp.dot(h, wa1[...], preferred_element_type=jnp.float32) + ba1[...])
    s2 = _lrelu(jnp.dot(s1, wa2[...], preferred_element_type=jnp.float32) + ba2[...])
    logits = jnp.dot(s2, wa3[...], preferred_element_type=jnp.float32) + ba3[...]
    masked = jnp.where(nbr > 0, logits, -1e9)
    m = jnp.max(masked)
    e = jnp.exp(masked - m)
    probs_ref[...] = e / jnp.sum(e)

    g = jnp.tanh(gcn(x_ref[...], wk1[...], bk1[...]))
    g = jnp.tanh(gcn(g, wk2[...], bk2[...]))
    g_ref[...] = jnp.tanh(gcn(g, wk3[...], bk3[...]))


HEAD_BLOCKS = 32
HEAD_ROWS = (N * 256) // HEAD_BLOCKS


def _head_body(gflat, w1, b1, w2, b2, w3, b3, w4, b4, out_ref, acc):
    k = pl.program_id(0)

    @pl.when(k == 0)
    def _init():
        acc[...] = jnp.zeros_like(acc)

    acc[...] += jnp.dot(gflat[...], w1[...], preferred_element_type=jnp.float32)

    @pl.when(k == HEAD_BLOCKS - 1)
    def _fini():
        v = _lrelu(acc[...] + b1[...])
        v = _lrelu(jnp.dot(v, w2[...], preferred_element_type=jnp.float32) + b2[...])
        v = _lrelu(jnp.dot(v, w3[...], preferred_element_type=jnp.float32) + b3[...])
        out_ref[...] = jnp.dot(v, w4[...], preferred_element_type=jnp.float32) + b4[...]


def kernel(x, edge_index, curr_vertex_index,
           Wc1, bc1, Wc2, bc2, Wc3, bc3,
           Wa1, ba1, Wa2, ba2, Wa3, ba3,
           Wk1, bk1, Wk2, bk2, Wk3, bk3,
           Wd1, bd1, Wd2, bd2, Wd3, bd3, Wd4, bd4):
    info = plsc.get_sparse_core_info()
    num_cores, num_subcores = info.num_cores, info.num_subcores

    src = edge_index[0]
    dst = edge_index[1]
    zero = jnp.zeros(((N * N) // num_subcores,), jnp.float32)

    cparts = _make_count_kernel(num_cores, num_subcores)(src, dst, zero)
    cparts = cparts.reshape(num_cores, N, N)
    if num_cores != 2:
        cparts = jnp.stack([cparts[0::2].sum(0), cparts[1::2].sum(0)])

    onehot = (jnp.arange(N, dtype=jnp.int32) ==
              jnp.asarray(curr_vertex_index, jnp.int32)).astype(jnp.float32)
    onehot = onehot.reshape(1, N)

    r = lambda b: b.reshape(1, -1)
    probs2, g = pl.pallas_call(
        _gnn_body,
        out_shape=(jax.ShapeDtypeStruct((N, 1), jnp.float32),
                   jax.ShapeDtypeStruct((N, 256), jnp.float32)),
        scratch_shapes=[pltpu.VMEM((N, N), jnp.float32)],
    )(cparts, x, onehot,
      Wc1, r(bc1), Wc2, r(bc2), Wc3, r(bc3),
      Wk1, r(bk1), Wk2, r(bk2), Wk3, r(bk3),
      Wa1, r(ba1), Wa2, r(ba2), Wa3, r(ba3))

    gflat = g.reshape(1, N * 256)
    value2 = pl.pallas_call(
        _head_body,
        grid=(HEAD_BLOCKS,),
        in_specs=[
            pl.BlockSpec((1, HEAD_ROWS), lambda k: (0, k)),
            pl.BlockSpec((HEAD_ROWS, 256), lambda k: (k, 0)),
            pl.BlockSpec((1, 256), lambda k: (0, 0)),
            pl.BlockSpec((256, 256), lambda k: (0, 0)),
            pl.BlockSpec((1, 256), lambda k: (0, 0)),
            pl.BlockSpec((256, 256), lambda k: (0, 0)),
            pl.BlockSpec((1, 256), lambda k: (0, 0)),
            pl.BlockSpec((256, 1), lambda k: (0, 0)),
            pl.BlockSpec((1, 1), lambda k: (0, 0)),
        ],
        out_specs=pl.BlockSpec((1, 1), lambda k: (0, 0)),
        out_shape=jax.ShapeDtypeStruct((1, 1), jnp.float32),
        scratch_shapes=[pltpu.VMEM((1, 256), jnp.float32)],
    )(gflat, Wd1, r(bd1), Wd2, r(bd2), Wd3, r(bd3), Wd4, r(bd4))

    return probs2.reshape(N), value2.reshape(1)


# speedup vs baseline: 14.5804x; 1.0600x over previous
import functools

import jax
import jax.numpy as jnp
from jax import lax
from jax.experimental import pallas as pl
from jax.experimental.pallas import tpu as pltpu
from jax.experimental.pallas import tpu_sc as plsc

N = 1024
E = 32768
EMB = 256
ALPHA = 0.1

NBLK = 32
BLK_ROWS = (N * EMB) // NBLK
NBUF = 4
NODES_PER_BLK = BLK_ROWS // EMB


def _make_count_kernel(num_cores, num_subcores):
    NW = num_cores * num_subcores
    EPW = E // NW
    CHUNK = 128
    NCHUNK = EPW // CHUNK
    WPT = (N * N) // num_subcores

    mesh = plsc.VectorSubcoreMesh(core_axis_name="c", subcore_axis_name="s")

    @functools.partial(
        pl.kernel,
        mesh=mesh,
        out_type=jax.ShapeDtypeStruct((num_cores, num_subcores, WPT), jnp.float32),
        scratch_types=[
            pltpu.VMEM((EPW,), jnp.int32),
            pltpu.VMEM((EPW,), jnp.int32),
            pltpu.VMEM((NCHUNK, CHUNK), jnp.int32),
            pltpu.VMEM((CHUNK,), jnp.float32),
            pltpu.VMEM_SHARED((N * N,), jnp.float32),
        ],
    )
    def count_kernel(src_hbm, dst_hbm, zero_hbm, out_hbm,
                     src_v, dst_v, idx_v, ones_v, c_sh):
        c = lax.axis_index("c")
        s = lax.axis_index("s")
        base = (c * num_subcores + s) * EPW

        pltpu.sync_copy(zero_hbm, c_sh.at[pl.ds(s * WPT, WPT)])
        pltpu.sync_copy(src_hbm.at[pl.ds(base, EPW)], src_v)
        pltpu.sync_copy(dst_hbm.at[pl.ds(base, EPW)], dst_v)

        for i in range(CHUNK // 16):
            ones_v[pl.ds(i * 16, 16)] = jnp.ones((16,), jnp.float32)
        for k in range(EPW // 16):
            sv = src_v[pl.ds(k * 16, 16)]
            dv = dst_v[pl.ds(k * 16, 16)]
            idx_v[(k * 16) // CHUNK, pl.ds((k * 16) % CHUNK, 16)] = dv * N + sv

        plsc.subcore_barrier()
        for j in range(NCHUNK):
            pltpu.sync_copy(ones_v, c_sh.at[idx_v.at[j]], add=True)
        plsc.subcore_barrier()
        pltpu.sync_copy(c_sh.at[pl.ds(s * WPT, WPT)], out_hbm.at[c, s])

    return count_kernel


def _lrelu(v):
    return jnp.where(v > 0, v, ALPHA * v)


def _fused_body(cp, x_ref, oh_ref,
                wc1, bc1, wc2, bc2, wc3, bc3,
                wk1, bk1, wk2, bk2, wk3, bk3,
                wa1, ba1, wa2, ba2, wa3, ba3,
                wd1_hbm, bd1, wd2, bd2, wd3, bd3, wd4, bd4,
                probs_ref, val_ref,
                c_scr, ring, sems):
    for b in range(NBUF):
        pltpu.make_async_copy(
            wd1_hbm.at[pl.ds(b * BLK_ROWS, BLK_ROWS), :],
            ring.at[b], sems.at[b]).start()

    c_scr[...] = cp[0] + cp[1]
    C = c_scr[...]
    deg = jnp.sum(C, axis=1, keepdims=True) + 1.0
    dinv = lax.rsqrt(deg)
    nbr = jnp.sum(C * oh_ref[...], axis=1, keepdims=True)

    def gcn(h, W, b):
        u = jnp.dot(h, W, preferred_element_type=jnp.float32)
        us = u * dinv
        agg = jnp.dot(C, us, preferred_element_type=jnp.float32) + us
        return agg * dinv + b

    h = jnp.tanh(gcn(x_ref[...], wc1[...], bc1[...]))
    h = jnp.tanh(gcn(h, wc2[...], bc2[...]))
    h = jnp.tanh(gcn(h, wc3[...], bc3[...]))
    s1 = _lrelu(jnp.dot(h, wa1[...], preferred_element_type=jnp.float32) + ba1[...])
    s2 = _lrelu(jnp.dot(s1, wa2[...], preferred_element_type=jnp.float32) + ba2[...])
    logits = jnp.dot(s2, wa3[...], preferred_element_type=jnp.float32) + ba3[...]
    masked = jnp.where(nbr > 0, logits, -1e9)
    m = jnp.max(masked)
    e = jnp.exp(masked - m)
    probs_ref[...] = e / jnp.sum(e)

    g = jnp.tanh(gcn(x_ref[...], wk1[...], bk1[...]))
    g = jnp.tanh(gcn(g, wk2[...], bk2[...]))
    g = jnp.tanh(gcn(g, wk3[...], bk3[...]))
    gflat = g.reshape(1, N * EMB)

    acc = jnp.zeros((1, EMB), jnp.float32)
    for k in range(NBLK):
        b = k % NBUF
        pltpu.make_async_copy(
            wd1_hbm.at[pl.ds(k * BLK_ROWS, BLK_ROWS), :],
            ring.at[b], sems.at[b]).wait()
        gs = gflat[:, k * BLK_ROWS:(k + 1) * BLK_ROWS]
        acc = acc + jnp.dot(gs, ring[b], preferred_element_type=jnp.float32)
        nk = k + NBUF
        if nk < NBLK:
            pltpu.make_async_copy(
                wd1_hbm.at[pl.ds(nk * BLK_ROWS, BLK_ROWS), :],
                ring.at[b], sems.at[b]).start()

    v = _lrelu(acc + bd1[...])
    v = _lrelu(jnp.dot(v, wd2[...], preferred_element_type=jnp.float32) + bd2[...])
    v = _lrelu(jnp.dot(v, wd3[...], preferred_element_type=jnp.float32) + bd3[...])
    val_ref[...] = jnp.dot(v, wd4[...], preferred_element_type=jnp.float32) + bd4[...]


def kernel(x, edge_index, curr_vertex_index,
           Wc1, bc1, Wc2, bc2, Wc3, bc3,
           Wa1, ba1, Wa2, ba2, Wa3, ba3,
           Wk1, bk1, Wk2, bk2, Wk3, bk3,
           Wd1, bd1, Wd2, bd2, Wd3, bd3, Wd4, bd4):
    info = plsc.get_sparse_core_info()
    num_cores, num_subcores = info.num_cores, info.num_subcores

    src = edge_index[0]
    dst = edge_index[1]
    zero = jnp.zeros(((N * N) // num_subcores,), jnp.float32)

    cparts = _make_count_kernel(num_cores, num_subcores)(src, dst, zero)
    cparts = cparts.reshape(num_cores, N, N)
    if num_cores != 2:
        cparts = jnp.stack([cparts[0::2].sum(0), cparts[1::2].sum(0)])

    onehot = (jnp.arange(N, dtype=jnp.int32) ==
              jnp.asarray(curr_vertex_index, jnp.int32)).astype(jnp.float32)
    onehot = onehot.reshape(1, N)

    r = lambda b: b.reshape(1, -1)
    vmem = pl.BlockSpec(memory_space=pltpu.MemorySpace.VMEM)
    nin = 21
    probs2, value2 = pl.pallas_call(
        _fused_body,
        in_specs=[vmem] * nin + [pl.BlockSpec(memory_space=pl.ANY)] + [vmem] * 7,
        out_shape=(jax.ShapeDtypeStruct((N, 1), jnp.float32),
                   jax.ShapeDtypeStruct((1, 1), jnp.float32)),
        scratch_shapes=[
            pltpu.VMEM((N, N), jnp.float32),
            pltpu.VMEM((NBUF, BLK_ROWS, EMB), jnp.float32),
            pltpu.SemaphoreType.DMA((NBUF,)),
        ],
    )(cparts, x, onehot,
      Wc1, r(bc1), Wc2, r(bc2), Wc3, r(bc3),
      Wk1, r(bk1), Wk2, r(bk2), Wk3, r(bk3),
      Wa1, r(ba1), Wa2, r(ba2), Wa3, r(ba3),
      Wd1, r(bd1), Wd2, r(bd2), Wd3, r(bd3), Wd4, r(bd4))

    return probs2.reshape(N), value2.reshape(1)
